# baseline (speedup 1.0000x reference)
import jax
import jax.numpy as jnp
from jax import lax
from jax.experimental import pallas as pl
from jax.experimental.pallas import tpu as pltpu
from jax.experimental.pallas import tpu_sc as plsc

N = 65536
DEG = 64
COLS = 64
L = 16
W = COLS // 2
NC = 2
NS = 16
NW = NC * NS
ROWS_W = N // NW
R = 16
E = R * DEG
CHUNK = 128
NCH = E // CHUNK
NB = ROWS_W // R
NACC = 4


def _body(inp_h, cols_h, vals_h, bias_h, out_h,
          idx_b, vals_b, rows_b, out_b, bias_b,
          sem_g0, sem_g1, sem_i0, sem_i1, sem_i2, sem_i3,
          sem_v0, sem_v1, sem_v2, sem_v3, sem_o0, sem_o1):
  sem_g = (sem_g0, sem_g1)
  sem_i = (sem_i0, sem_i1, sem_i2, sem_i3)
  sem_v = (sem_v0, sem_v1, sem_v2, sem_v3)
  sem_o = (sem_o0, sem_o1)
  wid = lax.axis_index("s") * NC + lax.axis_index("c")
  row0 = wid * ROWS_W
  e0 = row0 * DEG

  pltpu.sync_copy(bias_h.at[pl.ds(row0, ROWS_W)], bias_b.at[pl.ds(0, ROWS_W)])

  def fire_meta(g, s4):
    pltpu.async_copy(cols_h.at[pl.ds(e0 + g * E, E)], idx_b.at[s4], sem_i[s4])
    pltpu.async_copy(vals_h.at[pl.ds(e0 + g * E, E)], vals_b.at[s4], sem_v[s4])

  def fire_gathers(g, s4, s2):
    pltpu.make_async_copy(cols_h.at[pl.ds(0, E)], idx_b.at[s4],
                          sem_i[s4]).wait()
    for k in range(NCH):
      pltpu.async_copy(inp_h.at[idx_b.at[s4, pl.ds(k * CHUNK, CHUNK)]],
                       rows_b.at[s2, pl.ds(k * CHUNK, CHUNK)], sem_g[s2])

  def compute_block(g, s4, s2):
    pltpu.make_async_copy(inp_h.at[pl.ds(0, E)], rows_b.at[s2],
                          sem_g[s2]).wait()
    pltpu.make_async_copy(vals_h.at[pl.ds(0, E)], vals_b.at[s4],
                          sem_v[s4]).wait()

    def row_body(ri, carry):
      for rr in range(2):
        r = 2 * ri + rr
        bias_s = bias_b[pl.ds(g * R + r, L)][0]
        ebase = r * DEG
        zero = jnp.zeros((2 * L,), jnp.bfloat16)
        acc = [[zero for _ in range(NACC)] for _ in range(2)]
        for t in range(DEG // L):
          vv = vals_b[s4, pl.ds(ebase + t * L, L)]
          pv = plsc.bitcast(
              plsc.pack(vv, vv, format=plsc.PackFormat.INTERLEAVED),
              jnp.float32)
          for j in range(L):
            e = ebase + t * L + j
            vbf = plsc.bitcast(jnp.full((L,), pv[j], jnp.float32),
                               jnp.bfloat16)
            k = (t * L + j) % NACC
            for h in range(2):
              x = plsc.bitcast(rows_b[s2, e, pl.ds(h * L, L)], jnp.bfloat16)
              acc[h][k] = acc[h][k] + x * vbf
        for h in range(2):
          s = (acc[h][0] + acc[h][1]) + (acc[h][2] + acc[h][3])
          evens, odds = plsc.unpack(s, format=plsc.PackFormat.INTERLEAVED)
          out_b[s2, r, pl.ds(h * 2 * L, L)] = evens + bias_s
          out_b[s2, r, pl.ds(h * 2 * L + L, L)] = odds + bias_s
      return carry
    lax.fori_loop(0, R // 2, row_body, 0)
    pltpu.async_copy(out_b.at[s2], out_h.at[pl.ds(row0 + g * R, R)],
                     sem_o[s2])

  fire_meta(0, 0)
  fire_meta(1, 1)
  fire_gathers(0, 0, 0)

  def outer(t, carry):
    for b in range(4):
      g = 4 * t + b
      s4 = b
      s2 = b % 2

      @pl.when(g + 2 < NB)
      def _():
        fire_meta(g + 2, (s4 + 2) % 4)

      @pl.when(g + 1 < NB)
      def _():
        fire_gathers(g + 1, (s4 + 1) % 4, 1 - s2)

      @pl.when(g >= 2)
      def _():
        pltpu.make_async_copy(out_b.at[s2], out_h.at[pl.ds(row0, R)],
                              sem_o[s2]).wait()

      compute_block(g, s4, s2)
    return carry

  lax.fori_loop(0, NB // 4, outer, 0)

  for s2 in range(2):
    pltpu.make_async_copy(out_b.at[s2], out_h.at[pl.ds(row0, R)],
                          sem_o[s2]).wait()


PR = 256


def _pack_body(inp_h, out_h, in_b, pk_b, sem_pi0, sem_pi1, sem_po0, sem_po1):
  sem_pi = (sem_pi0, sem_pi1)
  sem_po = (sem_po0, sem_po1)
  wid = lax.axis_index("s") * NC + lax.axis_index("c")
  row0 = wid * ROWS_W
  NPB = ROWS_W // PR

  pltpu.async_copy(inp_h.at[pl.ds(row0, PR)], in_b.at[0], sem_pi[0])

  def blk_body(t, carry):
    for b in range(2):
      blk = 2 * t + b
      base = row0 + blk * PR

      @pl.when(blk + 1 < NPB)
      def _():
        pltpu.async_copy(inp_h.at[pl.ds(base + PR, PR)], in_b.at[1 - b],
                         sem_pi[1 - b])

      pltpu.make_async_copy(inp_h.at[pl.ds(0, PR)], in_b.at[b],
                            sem_pi[b]).wait()

      @pl.when(blk >= 2)
      def _():
        pltpu.make_async_copy(pk_b.at[b], out_h.at[pl.ds(row0, PR)],
                              sem_po[b]).wait()

      def row_body(r, c2):
        for h in range(2):
          a = in_b[b, r, pl.ds(h * 2 * L, L)]
          c = in_b[b, r, pl.ds(h * 2 * L + L, L)]
          packed = plsc.pack(a, c, format=plsc.PackFormat.INTERLEAVED)
          pk_b[b, r, pl.ds(h * L, L)] = plsc.bitcast(packed, jnp.float32)
        return c2
      lax.fori_loop(0, PR, row_body, 0)
      pltpu.async_copy(pk_b.at[b], out_h.at[pl.ds(base, PR)], sem_po[b])
    return carry
  lax.fori_loop(0, NPB // 2, blk_body, 0)

  for b in range(2):
    pltpu.make_async_copy(pk_b.at[b], out_h.at[pl.ds(row0, PR)],
                          sem_po[b]).wait()


@jax.jit
def _run(inp, cols, vals, bias):
  mesh = plsc.VectorSubcoreMesh(core_axis_name="c", subcore_axis_name="s",
                                num_cores=NC, num_subcores=NS)
  params = pltpu.CompilerParams(use_tc_tiling_on_sc=False,
                                needs_layout_passes=False)
  inp32 = pl.kernel(
      _pack_body,
      out_type=jax.ShapeDtypeStruct((N, W), jnp.float32),
      mesh=mesh,
      compiler_params=pltpu.CompilerParams(needs_layout_passes=False),
      scratch_types=[
          pltpu.VMEM((2, PR, COLS), jnp.float32),
          pltpu.VMEM((2, PR, W), jnp.float32),
      ] + [pltpu.SemaphoreType.DMA] * 4,
  )(inp)
  return pl.kernel(
      _body,
      out_type=jax.ShapeDtypeStruct((N, COLS), jnp.float32),
      mesh=mesh,
      compiler_params=params,
      scratch_types=[
          pltpu.VMEM((4, E), jnp.int32),
          pltpu.VMEM((4, E), jnp.float32),
          pltpu.VMEM((2, E, W), jnp.float32),
          pltpu.VMEM((2, R, COLS), jnp.float32),
          pltpu.VMEM((ROWS_W + L,), jnp.float32),
      ] + [pltpu.SemaphoreType.DMA] * 12,
  )(inp32, cols, vals, bias)


def kernel(inp, rows, cols, vals, bias):
  del rows
  return _run(inp, cols, vals, bias)


# speedup vs baseline: 1.0100x; 1.0100x over previous
import jax
import jax.numpy as jnp
from jax import lax
from jax.experimental import pallas as pl
from jax.experimental.pallas import tpu as pltpu
from jax.experimental.pallas import tpu_sc as plsc

N = 65536
DEG = 64
COLS = 64
L = 16
W = COLS // 2
NC = 2
NS = 16
NW = NC * NS
ROWS_W = N // NW
R = 16
E = R * DEG
CHUNK = 128
NCH = E // CHUNK
NB = ROWS_W // R
NACC = 4


def _body(inp_h, cols_h, vals_h, bias_h, out_h,
          idx_b, vals_b, rows_b, out_b, bias_b,
          sem_g0, sem_g1, sem_i0, sem_i1, sem_i2, sem_i3,
          sem_v0, sem_v1, sem_v2, sem_v3, sem_o0, sem_o1):
  sem_g = (sem_g0, sem_g1)
  sem_i = (sem_i0, sem_i1, sem_i2, sem_i3)
  sem_v = (sem_v0, sem_v1, sem_v2, sem_v3)
  sem_o = (sem_o0, sem_o1)
  wid = lax.axis_index("s") * NC + lax.axis_index("c")
  row0 = wid * ROWS_W
  e0 = row0 * DEG

  pltpu.sync_copy(bias_h.at[pl.ds(row0, ROWS_W)], bias_b.at[pl.ds(0, ROWS_W)])

  def fire_meta(g, s4):
    pltpu.async_copy(cols_h.at[pl.ds(e0 + g * E, E)], idx_b.at[s4], sem_i[s4])
    pltpu.async_copy(vals_h.at[pl.ds(e0 + g * E, E)], vals_b.at[s4], sem_v[s4])

  def fire_gathers(g, s4, s2):
    pltpu.make_async_copy(cols_h.at[pl.ds(0, E)], idx_b.at[s4],
                          sem_i[s4]).wait()
    for k in range(NCH):
      pltpu.async_copy(inp_h.at[idx_b.at[s4, pl.ds(k * CHUNK, CHUNK)]],
                       rows_b.at[s2, pl.ds(k * CHUNK, CHUNK)], sem_g[s2])

  def compute_block(g, s4, s2):
    pltpu.make_async_copy(inp_h.at[pl.ds(0, E)], rows_b.at[s2],
                          sem_g[s2]).wait()
    pltpu.make_async_copy(vals_h.at[pl.ds(0, E)], vals_b.at[s4],
                          sem_v[s4]).wait()

    def row_body(ri, carry):
      for rr in range(2):
        r = 2 * ri + rr
        bias_s = bias_b[pl.ds(g * R + r, L)][0]
        ebase = r * DEG
        zero = jnp.zeros((2 * L,), jnp.bfloat16)
        acc = [[zero for _ in range(NACC)] for _ in range(2)]
        for t in range(DEG // L):
          vv = vals_b[s4, pl.ds(ebase + t * L, L)]
          pv = plsc.bitcast(
              plsc.pack(vv, vv, format=plsc.PackFormat.INTERLEAVED),
              jnp.float32)
          for j in range(L):
            e = ebase + t * L + j
            vbf = plsc.bitcast(jnp.full((L,), pv[j], jnp.float32),
                               jnp.bfloat16)
            k = (t * L + j) % NACC
            for h in range(2):
              x = plsc.bitcast(rows_b[s2, e, pl.ds(h * L, L)], jnp.bfloat16)
              acc[h][k] = acc[h][k] + x * vbf
        for h in range(2):
          s = (acc[h][0] + acc[h][1]) + (acc[h][2] + acc[h][3])
          evens, odds = plsc.unpack(s, format=plsc.PackFormat.INTERLEAVED)
          out_b[s2, r, pl.ds(h * 2 * L, L)] = evens + bias_s
          out_b[s2, r, pl.ds(h * 2 * L + L, L)] = odds + bias_s
      return carry
    lax.fori_loop(0, R // 2, row_body, 0)
    pltpu.async_copy(out_b.at[s2], out_h.at[pl.ds(row0 + g * R, R)],
                     sem_o[s2])

  fire_meta(0, 0)
  fire_meta(1, 1)
  fire_gathers(0, 0, 0)

  def outer(t, carry):
    for b in range(4):
      g = 4 * t + b
      s4 = b
      s2 = b % 2

      @pl.when(g + 2 < NB)
      def _():
        fire_meta(g + 2, (s4 + 2) % 4)

      @pl.when(g + 1 < NB)
      def _():
        fire_gathers(g + 1, (s4 + 1) % 4, 1 - s2)

      @pl.when(g >= 2)
      def _():
        pltpu.make_async_copy(out_b.at[s2], out_h.at[pl.ds(row0, R)],
                              sem_o[s2]).wait()

      compute_block(g, s4, s2)
    return carry

  lax.fori_loop(0, NB // 4, outer, 0)

  for s2 in range(2):
    pltpu.make_async_copy(out_b.at[s2], out_h.at[pl.ds(row0, R)],
                          sem_o[s2]).wait()


PR = 256


def _pack_body(inp_h, out_h, in_b, pk_b, sem_pi0, sem_pi1, sem_po0, sem_po1):
  sem_pi = (sem_pi0, sem_pi1)
  sem_po = (sem_po0, sem_po1)
  wid = lax.axis_index("s") * NC + lax.axis_index("c")
  row0 = wid * ROWS_W
  NPB = ROWS_W // PR

  pltpu.async_copy(inp_h.at[pl.ds(row0, PR)], in_b.at[0], sem_pi[0])

  def blk_body(t, carry):
    for b in range(2):
      blk = 2 * t + b
      base = row0 + blk * PR

      @pl.when(blk + 1 < NPB)
      def _():
        pltpu.async_copy(inp_h.at[pl.ds(base + PR, PR)], in_b.at[1 - b],
                         sem_pi[1 - b])

      pltpu.make_async_copy(inp_h.at[pl.ds(0, PR)], in_b.at[b],
                            sem_pi[b]).wait()

      @pl.when(blk >= 2)
      def _():
        pltpu.make_async_copy(pk_b.at[b], out_h.at[pl.ds(row0, PR)],
                              sem_po[b]).wait()

      def row_body(r, c2):
        for h in range(2):
          a = in_b[b, r, pl.ds(h * 2 * L, L)]
          c = in_b[b, r, pl.ds(h * 2 * L + L, L)]
          packed = plsc.pack(a, c, format=plsc.PackFormat.INTERLEAVED)
          pk_b[b, r, pl.ds(h * L, L)] = plsc.bitcast(packed, jnp.float32)
        return c2
      lax.fori_loop(0, PR, row_body, 0)
      pltpu.async_copy(pk_b.at[b], out_h.at[pl.ds(base, PR)], sem_po[b])
    return carry
  lax.fori_loop(0, NPB // 2, blk_body, 0)

  for b in range(2):
    pltpu.make_async_copy(pk_b.at[b], out_h.at[pl.ds(row0, PR)],
                          sem_po[b]).wait()


@jax.jit
def _run(inp, cols, vals, bias):
  mesh = plsc.VectorSubcoreMesh(core_axis_name="c", subcore_axis_name="s",
                                num_cores=NC, num_subcores=NS)
  params = pltpu.CompilerParams(use_tc_tiling_on_sc=False,
                                needs_layout_passes=False)
  inp32 = pl.kernel(
      _pack_body,
      out_type=jax.ShapeDtypeStruct((N, W), jnp.float32),
      mesh=mesh,
      compiler_params=params,
      scratch_types=[
          pltpu.VMEM((2, PR, COLS), jnp.float32),
          pltpu.VMEM((2, PR, W), jnp.float32),
      ] + [pltpu.SemaphoreType.DMA] * 4,
  )(inp)
  return pl.kernel(
      _body,
      out_type=jax.ShapeDtypeStruct((N, COLS), jnp.float32),
      mesh=mesh,
      compiler_params=params,
      scratch_types=[
          pltpu.VMEM((4, E), jnp.int32),
          pltpu.VMEM((4, E), jnp.float32),
          pltpu.VMEM((2, E, W), jnp.float32),
          pltpu.VMEM((2, R, COLS), jnp.float32),
          pltpu.VMEM((ROWS_W + L,), jnp.float32),
      ] + [pltpu.SemaphoreType.DMA] * 12,
  )(inp32, cols, vals, bias)


def kernel(inp, rows, cols, vals, bias):
  del rows
  return _run(inp, cols, vals, bias)
